# baseline (speedup 1.0000x reference)
import functools

import jax
import jax.numpy as jnp
from jax.experimental import pallas as pl
from jax.experimental.pallas import tpu as pltpu

_B, _D, _H, _W = 32, 256, 32, 32
_HW = _H * _W
_K = 1024
_BETA = 0.2


def _vq_kernel(z_ref, cb_ref, q_ref, loss_ref):
    z_b = z_ref[0].reshape(_D, _HW)
    cb = cb_ref[...]

    zsq = jnp.sum(z_b * z_b, axis=0, keepdims=True)
    cbsq = jnp.sum(cb * cb, axis=1, keepdims=True)
    m = jax.lax.dot_general(
        cb, z_b, (((1,), (0,)), ((), ())),
        preferred_element_type=jnp.float32,
        precision=jax.lax.Precision.DEFAULT,
    )
    dist = (zsq + cbsq) - 2.0 * m

    minval = jnp.min(dist, axis=0, keepdims=True)
    iota_k = jax.lax.broadcasted_iota(jnp.int32, (_K, _HW), 0)
    masked = jnp.where(dist == minval, iota_k, _K)
    idx = jnp.min(masked, axis=0, keepdims=True)
    onehot = (iota_k == idx).astype(jnp.float32)

    q = jax.lax.dot_general(
        cb, onehot, (((0,), (0,)), ((), ())),
        preferred_element_type=jnp.float32,
        precision=jax.lax.Precision.DEFAULT,
    )
    q_ref[0] = q.reshape(_D, _H, _W)

    r = z_b - q
    loss_ref[0, 0, 0] = jnp.sum(r * r)


@functools.partial(jax.jit, static_argnames=())
def kernel(z, codebook_weight):
    b, d, h, w = z.shape
    quantized, loss_parts = pl.pallas_call(
        _vq_kernel,
        grid=(b,),
        in_specs=[
            pl.BlockSpec((1, d, h, w), lambda i: (i, 0, 0, 0)),
            pl.BlockSpec((_K, d), lambda i: (0, 0)),
        ],
        out_specs=[
            pl.BlockSpec((1, d, h, w), lambda i: (i, 0, 0, 0)),
            pl.BlockSpec((1, 1, 1), lambda i: (i, 0, 0), memory_space=pltpu.SMEM),
        ],
        out_shape=[
            jax.ShapeDtypeStruct((b, d, h, w), jnp.float32),
            jax.ShapeDtypeStruct((b, 1, 1), jnp.float32),
        ],
        compiler_params=pltpu.CompilerParams(
            dimension_semantics=("parallel",),
        ),
    )(z, codebook_weight)
    total = jnp.sum(loss_parts)
    codebook_loss = total / (b * h * w * d)
    commitment_loss = _BETA * codebook_loss
    return (quantized, codebook_loss, commitment_loss)


# speedup vs baseline: 1.8338x; 1.8338x over previous
import functools

import jax
import jax.numpy as jnp
from jax import lax
from jax.experimental import pallas as pl
from jax.experimental.pallas import tpu as pltpu
from jax.experimental.pallas import tpu_sc as plsc

_B, _D, _H, _W = 32, 256, 32, 32
_HW = _H * _W
_K = 1024
_BETA = 0.2
_N = _B * _HW


def _argmin_kernel(z_ref, cb_ref, cbh_ref, idx_ref, loss_ref):
    z_b = z_ref[0]
    cb = cb_ref[...]
    cbh = cbh_ref[...]

    zsq = jnp.sum(z_b * z_b, axis=0, keepdims=True)
    cbsq = jnp.sum(cb * cb, axis=1, keepdims=True)
    m2 = jax.lax.dot_general(
        cbh, z_b.astype(jnp.bfloat16), (((1,), (0,)), ((), ())),
        preferred_element_type=jnp.float32,
    )
    dist = (zsq + cbsq) - m2

    minval = jnp.min(dist, axis=0, keepdims=True)
    iota_k = jax.lax.broadcasted_iota(
        jnp.int32, (_K, _HW), 0).astype(jnp.float32)
    masked = jnp.where(dist == minval, iota_k, jnp.float32(_K))
    idx = jnp.min(masked, axis=0, keepdims=True)
    idx_ref[0] = idx.astype(jnp.int32)
    loss_ref[0, 0, 0] = jnp.sum(minval)


def _transpose_kernel(rows_ref, q_ref):
    q_ref[0] = rows_ref[0].T


_CHUNK = 256


def _sc_gather(table, idx_flat):
    info = plsc.get_sparse_core_info()
    nc, ns = info.num_cores, info.num_subcores
    nw = nc * ns
    b_per_w = _N // nw
    n_chunks = b_per_w // _CHUNK
    mesh = plsc.VectorSubcoreMesh(core_axis_name="c", subcore_axis_name="s")

    @functools.partial(
        pl.kernel, mesh=mesh,
        out_type=jax.ShapeDtypeStruct((_N, _D), jnp.float32),
        scratch_types=[
            pltpu.VMEM((_CHUNK,), jnp.int32),
            pltpu.VMEM((_CHUNK, _D), jnp.float32),
            pltpu.SemaphoreType.DMA,
        ],
    )
    def k(table_hbm, idx_hbm, out_hbm, idx_v, rows_v, sem):
        wid = lax.axis_index("s") * nc + lax.axis_index("c")
        base = wid * b_per_w
        for c in range(n_chunks):
            off = base + c * _CHUNK
            pltpu.sync_copy(idx_hbm.at[pl.ds(off, _CHUNK)], idx_v)
            pltpu.async_copy(table_hbm.at[idx_v], rows_v, sem).wait()
            pltpu.sync_copy(rows_v, out_hbm.at[pl.ds(off, _CHUNK)])

    return k(table, idx_flat)


@functools.partial(jax.jit, static_argnames=())
def kernel(z, codebook_weight):
    b, d, h, w = z.shape
    z3 = z.reshape(b, d, h * w)
    cbh2 = (2.0 * codebook_weight).astype(jnp.bfloat16)
    table = codebook_weight.astype(jnp.bfloat16).astype(jnp.float32)

    idx3, loss_parts = pl.pallas_call(
        _argmin_kernel,
        grid=(b,),
        in_specs=[
            pl.BlockSpec((1, d, h * w), lambda i: (i, 0, 0)),
            pl.BlockSpec((_K, d), lambda i: (0, 0)),
            pl.BlockSpec((_K, d), lambda i: (0, 0)),
        ],
        out_specs=[
            pl.BlockSpec((1, 1, h * w), lambda i: (i, 0, 0)),
            pl.BlockSpec((1, 1, 1), lambda i: (i, 0, 0), memory_space=pltpu.SMEM),
        ],
        out_shape=[
            jax.ShapeDtypeStruct((b, 1, h * w), jnp.int32),
            jax.ShapeDtypeStruct((b, 1, 1), jnp.float32),
        ],
        compiler_params=pltpu.CompilerParams(
            dimension_semantics=("arbitrary",),
        ),
    )(z3, codebook_weight, cbh2)

    idx_flat = idx3.reshape(_N)
    rows = _sc_gather(table, idx_flat)

    q3 = pl.pallas_call(
        _transpose_kernel,
        grid=(b,),
        in_specs=[pl.BlockSpec((1, h * w, d), lambda i: (i, 0, 0))],
        out_specs=pl.BlockSpec((1, d, h * w), lambda i: (i, 0, 0)),
        out_shape=jax.ShapeDtypeStruct((b, d, h * w), jnp.float32),
        compiler_params=pltpu.CompilerParams(
            dimension_semantics=("arbitrary",),
        ),
    )(rows.reshape(b, h * w, d))

    quantized = q3.reshape(b, d, h, w)
    total = jnp.sum(loss_parts)
    codebook_loss = total / (b * h * w * d)
    commitment_loss = _BETA * codebook_loss
    return (quantized, codebook_loss, commitment_loss)


# speedup vs baseline: 2.6570x; 1.4489x over previous
import functools

import jax
import jax.numpy as jnp
from jax.experimental import pallas as pl
from jax.experimental.pallas import tpu as pltpu

_B, _D, _H, _W = 32, 256, 32, 32
_HW = _H * _W
_K = 1024
_BETA = 0.2


def _vq_kernel(z_hbm, cb_ref, cbh_ref, q_hbm, loss_ref,
               zbuf, qbuf, in_sem, out_sem):
    i = pl.program_id(0)
    nb = pl.num_programs(0)
    slot = jax.lax.rem(i, 2)
    nslot = jax.lax.rem(i + 1, 2)

    @pl.when(i == 0)
    def _():
        pltpu.make_async_copy(z_hbm.at[0], zbuf.at[0], in_sem.at[0]).start()

    @pl.when(i + 1 < nb)
    def _():
        pltpu.make_async_copy(
            z_hbm.at[i + 1], zbuf.at[nslot], in_sem.at[nslot]).start()

    pltpu.make_async_copy(z_hbm.at[i], zbuf.at[slot], in_sem.at[slot]).wait()

    @pl.when(i >= 2)
    def _():
        pltpu.make_async_copy(
            qbuf.at[slot], q_hbm.at[i - 2], out_sem.at[slot]).wait()

    z_b = zbuf[slot]
    cb = cb_ref[...]
    cbh = cbh_ref[...]

    zsq = jnp.sum(z_b * z_b, axis=0, keepdims=True)
    cbsq = jnp.sum(cb * cb, axis=1, keepdims=True)
    m2 = jax.lax.dot_general(
        cbh, z_b.astype(jnp.bfloat16), (((1,), (0,)), ((), ())),
        preferred_element_type=jnp.float32,
    )
    dist = (zsq + cbsq) - m2

    minval = jnp.min(dist, axis=0, keepdims=True)
    iota_k = jax.lax.broadcasted_iota(
        jnp.int32, (_K, _HW), 0).astype(jnp.float32)
    masked = jnp.where(dist == minval, iota_k, jnp.float32(_K))
    idx = jnp.min(masked, axis=0, keepdims=True)
    onehot = (iota_k == idx).astype(jnp.bfloat16)

    q = jax.lax.dot_general(
        cbh, onehot, (((0,), (0,)), ((), ())),
        preferred_element_type=jnp.float32,
    ) * 0.5
    qbuf[slot] = q

    loss_ref[0, 0, 0] = jnp.sum(minval)

    pltpu.make_async_copy(qbuf.at[slot], q_hbm.at[i], out_sem.at[slot]).start()

    @pl.when(i == nb - 1)
    def _():
        pltpu.make_async_copy(
            qbuf.at[nslot], q_hbm.at[i - 1], out_sem.at[nslot]).wait()
        pltpu.make_async_copy(
            qbuf.at[slot], q_hbm.at[i], out_sem.at[slot]).wait()


@functools.partial(jax.jit, static_argnames=())
def kernel(z, codebook_weight):
    b, d, h, w = z.shape
    z3 = z.reshape(b, d, h * w)
    cb_bf16 = (2.0 * codebook_weight).astype(jnp.bfloat16)
    q3, loss_parts = pl.pallas_call(
        _vq_kernel,
        grid=(b,),
        in_specs=[
            pl.BlockSpec(memory_space=pltpu.MemorySpace.HBM),
            pl.BlockSpec((_K, d), lambda i: (0, 0)),
            pl.BlockSpec((_K, d), lambda i: (0, 0)),
        ],
        out_specs=[
            pl.BlockSpec(memory_space=pltpu.MemorySpace.HBM),
            pl.BlockSpec((1, 1, 1), lambda i: (i, 0, 0), memory_space=pltpu.SMEM),
        ],
        out_shape=[
            jax.ShapeDtypeStruct((b, d, h * w), jnp.float32),
            jax.ShapeDtypeStruct((b, 1, 1), jnp.float32),
        ],
        scratch_shapes=[
            pltpu.VMEM((2, d, h * w), jnp.float32),
            pltpu.VMEM((2, d, h * w), jnp.float32),
            pltpu.SemaphoreType.DMA((2,)),
            pltpu.SemaphoreType.DMA((2,)),
        ],
        compiler_params=pltpu.CompilerParams(
            dimension_semantics=("arbitrary",),
        ),
    )(z3, codebook_weight, cb_bf16)
    quantized = q3.reshape(b, d, h, w)
    total = jnp.sum(loss_parts)
    codebook_loss = total / (b * h * w * d)
    commitment_loss = _BETA * codebook_loss
    return (quantized, codebook_loss, commitment_loss)
